# baseline (speedup 1.0000x reference)
import functools

import jax
import jax.numpy as jnp
from jax.experimental import pallas as pl

_B, _S, _D = 4, 8192, 768
_E, _K = 8, 2
_ALPHA = 0.1
_T = 4096
_SB = _S // _T


def _gate_body(h_ref, w_ref, i1_ref, i2_ref, w1_ref, w2_ref,
               cnt_ref, ssum_ref):
    g = pl.program_id(0)
    x = h_ref[...]
    w = w_ref[...]
    lg = jax.lax.dot_general(w, x, (((1,), (1,)), ((), ())),
                             preferred_element_type=jnp.float32)
    m = jnp.max(lg, axis=0, keepdims=True)
    p = jnp.exp(lg - m)
    z = jnp.sum(p, axis=0, keepdims=True)
    iota = jax.lax.broadcasted_iota(jnp.int32, (_E, _T), 0)
    idx1 = jnp.min(jnp.where(lg == m, iota, _E), axis=0, keepdims=True)
    l2 = jnp.where(iota == idx1, -jnp.inf, lg)
    m2 = jnp.max(l2, axis=0, keepdims=True)
    idx2 = jnp.min(jnp.where(l2 == m2, iota, _E), axis=0, keepdims=True)
    s1 = 1.0 / z
    s2 = jnp.exp(m2 - m) * s1
    denom = s1 + s2 + 1e-20
    i1_ref[...] = idx1
    i2_ref[...] = idx2
    w1_ref[...] = s1 / denom
    w2_ref[...] = s2 / denom
    sel = (jnp.where(iota == idx1, 1.0, 0.0)
           + jnp.where(iota == idx2, 1.0, 0.0))
    ones = jnp.ones((1, _T), jnp.float32)
    cntc = jax.lax.dot_general(ones, sel, (((1,), (1,)), ((), ())),
                               preferred_element_type=jnp.float32)
    s = p * s1
    ssumc = jax.lax.dot_general(ones, s, (((1,), (1,)), ((), ())),
                                preferred_element_type=jnp.float32)

    @pl.when(g == 0)
    def _init():
        cnt_ref[...] = jnp.zeros_like(cnt_ref)
        ssum_ref[...] = jnp.zeros_like(ssum_ref)

    b = g // _SB
    riota = jax.lax.broadcasted_iota(jnp.int32, (_B, _E), 0)
    sel_row = riota == b
    cnt_ref[...] += jnp.where(sel_row, cntc, 0.0)
    ssum_ref[...] += jnp.where(sel_row, ssumc, 0.0)


@functools.partial(jax.jit, static_argnames=())
def kernel(hidden_states, weight):
    batch, seq, dim = hidden_states.shape
    n = batch * seq
    grid = n // _T
    h2 = hidden_states.reshape(n, dim)
    out_shapes = (
        jax.ShapeDtypeStruct((1, n), jnp.int32),
        jax.ShapeDtypeStruct((1, n), jnp.int32),
        jax.ShapeDtypeStruct((1, n), jnp.float32),
        jax.ShapeDtypeStruct((1, n), jnp.float32),
        jax.ShapeDtypeStruct((batch, _E), jnp.float32),
        jax.ShapeDtypeStruct((batch, _E), jnp.float32),
    )
    row = pl.BlockSpec((1, _T), lambda g: (0, g))
    acc = pl.BlockSpec((batch, _E), lambda g: (0, 0))
    i1, i2, w1, w2, cnt, ssum = pl.pallas_call(
        _gate_body,
        grid=(grid,),
        in_specs=[
            pl.BlockSpec((_T, dim), lambda g: (g, 0)),
            pl.BlockSpec((_E, dim), lambda g: (0, 0)),
        ],
        out_specs=(row, row, row, row, acc, acc),
        out_shape=out_shapes,
    )(h2, weight)
    topk_idx = jnp.zeros((n, 2), jnp.int32)
    topk_weight = jnp.zeros((n, 2), jnp.float32)
    ce = cnt * (_E / (seq * _K))
    aux_loss = jnp.mean(jnp.sum(ce * (ssum / seq), axis=1)) * _ALPHA
    return (topk_idx, topk_weight, aux_loss)


# speedup vs baseline: 1.0207x; 1.0207x over previous
import functools

import jax
import jax.numpy as jnp
from jax.experimental import pallas as pl

_B, _S, _D = 4, 8192, 768
_E, _K = 8, 2
_ALPHA = 0.1
_T = 4096
_SB = _S // _T


def _gate_body(h_ref, w_ref, i_ref, wt_ref, aux_ref, cnt_ref, ssum_ref):
    g = pl.program_id(0)
    ng = pl.num_programs(0)
    x = h_ref[...]
    w = w_ref[...]
    lg = jax.lax.dot_general(w, x, (((1,), (1,)), ((), ())),
                             preferred_element_type=jnp.float32)
    m = jnp.max(lg, axis=0, keepdims=True)
    p = jnp.exp(lg - m)
    z = jnp.sum(p, axis=0, keepdims=True)
    iota = jax.lax.broadcasted_iota(jnp.int32, (_E, _T), 0)
    idx1 = jnp.min(jnp.where(lg == m, iota, _E), axis=0, keepdims=True)
    l2 = jnp.where(iota == idx1, -jnp.inf, lg)
    m2 = jnp.max(l2, axis=0, keepdims=True)
    idx2 = jnp.min(jnp.where(l2 == m2, iota, _E), axis=0, keepdims=True)
    s1 = 1.0 / z
    s2 = jnp.exp(m2 - m) * s1
    denom = s1 + s2 + 1e-20
    i_ref[...] = jnp.stack([idx1, idx2], axis=0)
    wt_ref[...] = jnp.stack([s1 / denom, s2 / denom], axis=0)
    sel = (jnp.where(iota == idx1, 1.0, 0.0)
           + jnp.where(iota == idx2, 1.0, 0.0))
    ones = jnp.ones((1, _T), jnp.float32)
    cntc = jax.lax.dot_general(ones, sel, (((1,), (1,)), ((), ())),
                               preferred_element_type=jnp.float32)
    s = p * s1
    ssumc = jax.lax.dot_general(ones, s, (((1,), (1,)), ((), ())),
                                preferred_element_type=jnp.float32)

    @pl.when(g == 0)
    def _init():
        cnt_ref[...] = jnp.zeros_like(cnt_ref)
        ssum_ref[...] = jnp.zeros_like(ssum_ref)

    b = g // _SB
    riota = jax.lax.broadcasted_iota(jnp.int32, (_B, _E), 0)
    sel_row = riota == b
    cnt = cnt_ref[...] + jnp.where(sel_row, cntc, 0.0)
    ssum = ssum_ref[...] + jnp.where(sel_row, ssumc, 0.0)
    cnt_ref[...] = cnt
    ssum_ref[...] = ssum

    @pl.when(g == ng - 1)
    def _finish():
        ce = cnt * (_E / (_S * _K))
        aux_ref[...] = jnp.sum(ce * (ssum / _S), axis=(0, 1),
                               keepdims=True) * (_ALPHA / _B)


@functools.partial(jax.jit, static_argnames=())
def kernel(hidden_states, weight):
    batch, seq, dim = hidden_states.shape
    n = batch * seq
    grid = n // _T
    h2 = hidden_states.reshape(n, dim)
    out_shapes = (
        jax.ShapeDtypeStruct((2, 1, n), jnp.int32),
        jax.ShapeDtypeStruct((2, 1, n), jnp.float32),
        jax.ShapeDtypeStruct((1, 1), jnp.float32),
        jax.ShapeDtypeStruct((batch, _E), jnp.float32),
        jax.ShapeDtypeStruct((batch, _E), jnp.float32),
    )
    row2 = pl.BlockSpec((2, 1, _T), lambda g: (0, 0, g))
    one = pl.BlockSpec((1, 1), lambda g: (0, 0))
    acc = pl.BlockSpec((batch, _E), lambda g: (0, 0))
    idx, wgt, aux, _, _ = pl.pallas_call(
        _gate_body,
        grid=(grid,),
        in_specs=[
            pl.BlockSpec((_T, dim), lambda g: (g, 0)),
            pl.BlockSpec((_E, dim), lambda g: (0, 0)),
        ],
        out_specs=(row2, row2, one, acc, acc),
        out_shape=out_shapes,
    )(h2, weight)
    topk_idx = idx.reshape(2, n).T
    topk_weight = wgt.reshape(2, n).T
    return (topk_idx, topk_weight, aux.reshape(()))
